# Initial kernel scaffold; baseline (speedup 1.0000x reference)
import jax
import jax.numpy as jnp
from jax.experimental import pallas as pl


def kernel(x, sparsity, mask):
    raise NotImplementedError("write your pallas kernel here")


# speedup vs baseline: 25.3071x; 25.3071x over previous
import functools

import jax
import jax.numpy as jnp
from jax import lax
from jax.experimental import pallas as pl
from jax.experimental.pallas import tpu as pltpu
from jax.experimental.pallas import tpu_sc as plsc

N = 4096 * 4096
NW = 32
CHUNK = N // NW
PIECE = 16384
NPIECES = CHUNK // PIECE
UNROLL = 4
BINS1 = 1 << 12
BINS2 = 1 << 12
BINS3 = 1 << 7

_mesh = plsc.VectorSubcoreMesh(core_axis_name="c", subcore_axis_name="s")
_sc_params = pltpu.CompilerParams(needs_layout_passes=False)


def _worker_id():
    return lax.axis_index("s") * 2 + lax.axis_index("c")


def _make_hist_kernel(nbins, bin_fn, masked):
    scratch = [
        pltpu.VMEM((PIECE,), jnp.int32),
        pltpu.VMEM((16 * nbins,), jnp.int32),
        pltpu.VMEM((nbins,), jnp.int32),
    ]
    if masked:
        scratch.append(pltpu.VMEM((16,), jnp.int32))

    def body(x_hbm, *rest):
        if masked:
            bsel_hbm, out_hbm, data_v, hist_v, merged_v, b_v = rest
        else:
            out_hbm, data_v, hist_v, merged_v = rest
        wid = _worker_id()

        zeros = jnp.zeros((16,), jnp.int32)

        def z_body(i, c):
            hist_v[pl.ds(i * 16, 16)] = zeros
            return c

        lax.fori_loop(0, nbins, z_body, 0)

        if masked:
            pltpu.sync_copy(bsel_hbm.at[pl.ds(0, 16)], b_v)
            bvec = b_v[...]
        ones = jnp.full((16,), 1, jnp.int32)
        lane = lax.broadcasted_iota(jnp.int32, (16,), 0)
        lane_offs = [((lane + u) & jnp.int32(15)) * nbins
                     for u in range(UNROLL)]

        def piece_body(p, carry):
            base = pl.multiple_of(wid * CHUNK + p * PIECE, PIECE)
            pltpu.sync_copy(x_hbm.at[pl.ds(base, PIECE)], data_v)

            def vec_body(i, c2):
                off = i * (16 * UNROLL)
                for u in range(UNROLL):
                    v = data_v[pl.ds(off + u * 16, 16)]
                    bits = v & jnp.int32(0x7FFFFFFF)
                    b, key = bin_fn(bits)
                    addr = b + lane_offs[u]
                    if masked:
                        plsc.addupdate_scatter(hist_v, [addr], ones,
                                               mask=key == bvec)
                    else:
                        plsc.addupdate_scatter(hist_v, [addr], ones)
                return c2

            lax.fori_loop(0, PIECE // (16 * UNROLL), vec_body, 0)
            return carry

        lax.fori_loop(0, NPIECES, piece_body, 0)

        def m_body(i, c):
            acc = hist_v[pl.ds(i * 16, 16)]
            for l in range(1, 16):
                acc = acc + hist_v[pl.ds(l * nbins + i * 16, 16)]
            merged_v[pl.ds(i * 16, 16)] = acc
            return c

        lax.fori_loop(0, nbins // 16, m_body, 0)
        pltpu.sync_copy(merged_v, out_hbm.at[wid])

    return pl.kernel(
        body,
        out_type=jax.ShapeDtypeStruct((NW, nbins), jnp.int32),
        mesh=_mesh,
        scratch_types=scratch,
        compiler_params=_sc_params,
    )


_hist1 = _make_hist_kernel(
    BINS1, lambda bits: (bits >> 19, None), masked=False)
_hist2 = _make_hist_kernel(
    BINS2, lambda bits: ((bits >> 7) & jnp.int32(0xFFF), bits >> 19),
    masked=True)
_hist3 = _make_hist_kernel(
    BINS3, lambda bits: (bits & jnp.int32(0x7F), bits >> 7), masked=True)


def _cumsum_rows(c):
    nrows = c.shape[0]
    k = 1
    while k < nrows:
        shifted = jnp.concatenate(
            [jnp.zeros((k, 1), jnp.int32), c[:-k]], axis=0)
        c = c + shifted
        k *= 2
    return c


def _cumsum_lanes(c):
    k = 1
    while k < 128:
        shifted = jnp.concatenate(
            [jnp.zeros((1, k), jnp.int32), c[:, :-k]], axis=1)
        c = c + shifted
        k *= 2
    return c


def _select_bin(h2d, r, nrows):
    rows = jnp.sum(h2d, axis=1, keepdims=True)
    cum_rows = _cumsum_rows(rows)
    lt = cum_rows < r
    row_idx = jnp.sum(lt.astype(jnp.int32))
    base = jnp.max(jnp.where(lt, cum_rows, 0))
    rowsel = lax.broadcasted_iota(jnp.int32, (nrows, 128), 0) == row_idx
    row = jnp.sum(jnp.where(rowsel, h2d, 0), axis=0, keepdims=True)
    cum_in = _cumsum_lanes(row)
    lt2 = (base + cum_in) < r
    col_idx = jnp.sum(lt2.astype(jnp.int32))
    prev = base + jnp.max(jnp.where(lt2, cum_in, 0))
    return row_idx * 128 + col_idx, prev


def _find1_body(h_ref, r_ref, out_ref):
    r = r_ref[0, 0]
    h = jnp.sum(h_ref[...], axis=0)
    b1, prev = _select_bin(h, r, BINS1 // 128)
    r2 = r - prev
    rowi = lax.broadcasted_iota(jnp.int32, (8, 128), 0)
    out_ref[...] = jnp.where(rowi == 0, b1, r2)


def _find2_body(h_ref, br_ref, out_ref):
    b1 = br_ref[0, 0]
    r2 = br_ref[1, 0]
    h = jnp.sum(h_ref[...], axis=0)
    b2, prev = _select_bin(h, r2, BINS2 // 128)
    b12 = b1 * BINS2 + b2
    r3 = r2 - prev
    rowi = lax.broadcasted_iota(jnp.int32, (8, 128), 0)
    out_ref[...] = jnp.where(rowi == 0, b12, r3)


def _find3_body(h_ref, br_ref, out_ref):
    b12 = br_ref[0, 0]
    r3 = br_ref[1, 0]
    h = jnp.sum(h_ref[...], axis=0)
    cum = _cumsum_lanes(h)
    b3 = jnp.sum((cum < r3).astype(jnp.int32))
    tbits = b12 * BINS3 + b3
    out_ref[...] = jnp.full((1, 1), lax.bitcast_convert_type(tbits, jnp.float32))


def _mask_body(x_ref, t_ref, out_ref):
    t = t_ref[0, 0]
    xv = x_ref[...]
    out_ref[...] = jnp.where(jnp.abs(xv) > t, xv, 0.0)


_find1 = pl.pallas_call(
    _find1_body, out_shape=jax.ShapeDtypeStruct((8, 128), jnp.int32))
_find2 = pl.pallas_call(
    _find2_body, out_shape=jax.ShapeDtypeStruct((8, 128), jnp.int32))
_find3 = pl.pallas_call(
    _find3_body, out_shape=jax.ShapeDtypeStruct((1, 1), jnp.float32))

_ROWS_PER_BLOCK = 256

_apply_mask = pl.pallas_call(
    _mask_body,
    grid=(4096 // _ROWS_PER_BLOCK,),
    in_specs=[
        pl.BlockSpec((_ROWS_PER_BLOCK, 4096), lambda i: (i, 0)),
        pl.BlockSpec((1, 1), lambda i: (0, 0)),
    ],
    out_specs=pl.BlockSpec((_ROWS_PER_BLOCK, 4096), lambda i: (i, 0)),
    out_shape=jax.ShapeDtypeStruct((4096, 4096), jnp.float32),
)


def kernel(x, sparsity, mask):
    del mask
    xi = lax.bitcast_convert_type(x, jnp.int32).reshape(-1)
    s = sparsity.reshape(())
    idx = jnp.clip((s * N - 1.0).astype(jnp.int32), 0, N - 1)
    r = (idx + 1).reshape(1, 1)
    h1 = _hist1(xi)
    br1 = _find1(h1.reshape(NW, BINS1 // 128, 128), r)
    h2 = _hist2(xi, br1.reshape(-1))
    br2 = _find2(h2.reshape(NW, BINS2 // 128, 128), br1)
    h3 = _hist3(xi, br2.reshape(-1))
    thr = _find3(h3.reshape(NW, 1, BINS3), br2)
    return _apply_mask(x, thr)


# speedup vs baseline: 26.2415x; 1.0369x over previous
import functools

import jax
import jax.numpy as jnp
from jax import lax
from jax.experimental import pallas as pl
from jax.experimental.pallas import tpu as pltpu
from jax.experimental.pallas import tpu_sc as plsc

N = 4096 * 4096
NW = 32
CHUNK = N // NW
PIECE = 16384
NPIECES = CHUNK // PIECE
UNROLL = 4
BINS1 = 1 << 12
BINS2 = 1 << 12
BINS3 = 1 << 7

_mesh = plsc.VectorSubcoreMesh(core_axis_name="c", subcore_axis_name="s")
_sc_params = pltpu.CompilerParams(needs_layout_passes=False)


def _worker_id():
    return lax.axis_index("s") * 2 + lax.axis_index("c")


def _make_hist_kernel(nbins, bin_fn, masked):
    stride = nbins + 1
    scratch = [
        pltpu.VMEM((PIECE,), jnp.float32),
        pltpu.VMEM((16 * stride,), jnp.int32),
        pltpu.VMEM((nbins,), jnp.int32),
    ]
    if masked:
        scratch.append(pltpu.VMEM((16,), jnp.int32))

    def body(x_hbm, *rest):
        if masked:
            bsel_hbm, out_hbm, data_v, hist_v, merged_v, b_v = rest
        else:
            out_hbm, data_v, hist_v, merged_v = rest
        wid = _worker_id()

        zeros = jnp.zeros((16,), jnp.int32)

        def z_body(i, c):
            hist_v[pl.ds(i * 16, 16)] = zeros
            return c

        lax.fori_loop(0, 16 * stride // 16, z_body, 0)

        if masked:
            pltpu.sync_copy(bsel_hbm.at[pl.ds(0, 16)], b_v)
            bvec = b_v[...]
        ones = jnp.full((16,), 1, jnp.int32)
        lane = lax.broadcasted_iota(jnp.int32, (16,), 0)
        lane_off = lane * stride

        def piece_body(p, carry):
            base = pl.multiple_of(wid * CHUNK + p * PIECE, PIECE)
            pltpu.sync_copy(x_hbm.at[pl.ds(base, PIECE)], data_v)

            def vec_body(i, c2):
                off = i * (16 * UNROLL)
                for u in range(UNROLL):
                    v = data_v[pl.ds(off + u * 16, 16)]
                    bits = plsc.bitcast(v, jnp.int32) & jnp.int32(0x7FFFFFFF)
                    b, key = bin_fn(bits)
                    addr = b + lane_off
                    if masked:
                        plsc.addupdate_scatter(hist_v, [addr], ones,
                                               mask=key == bvec)
                    else:
                        plsc.addupdate_scatter(hist_v, [addr], ones)
                return c2

            lax.fori_loop(0, PIECE // (16 * UNROLL), vec_body, 0)
            return carry

        lax.fori_loop(0, NPIECES, piece_body, 0)

        def m_body(i, c):
            acc = hist_v[pl.ds(i * 16, 16)]
            for l in range(1, 16):
                acc = acc + hist_v[pl.ds(l * stride + i * 16, 16)]
            merged_v[pl.ds(i * 16, 16)] = acc
            return c

        lax.fori_loop(0, nbins // 16, m_body, 0)
        pltpu.sync_copy(merged_v, out_hbm.at[wid])

    return pl.kernel(
        body,
        out_type=jax.ShapeDtypeStruct((NW, nbins), jnp.int32),
        mesh=_mesh,
        scratch_types=scratch,
        compiler_params=_sc_params,
    )


_hist1 = _make_hist_kernel(
    BINS1, lambda bits: (bits >> 19, None), masked=False)
_hist2 = _make_hist_kernel(
    BINS2, lambda bits: ((bits >> 7) & jnp.int32(0xFFF), bits >> 19),
    masked=True)
_hist3 = _make_hist_kernel(
    BINS3, lambda bits: (bits & jnp.int32(0x7F), bits >> 7), masked=True)


def _cumsum_rows(c):
    nrows = c.shape[0]
    k = 1
    while k < nrows:
        shifted = jnp.concatenate(
            [jnp.zeros((k, 1), jnp.int32), c[:-k]], axis=0)
        c = c + shifted
        k *= 2
    return c


def _cumsum_lanes(c):
    k = 1
    while k < 128:
        shifted = jnp.concatenate(
            [jnp.zeros((1, k), jnp.int32), c[:, :-k]], axis=1)
        c = c + shifted
        k *= 2
    return c


def _select_bin(h2d, r, nrows):
    rows = jnp.sum(h2d, axis=1, keepdims=True)
    cum_rows = _cumsum_rows(rows)
    lt = cum_rows < r
    row_idx = jnp.sum(lt.astype(jnp.int32))
    base = jnp.max(jnp.where(lt, cum_rows, 0))
    rowsel = lax.broadcasted_iota(jnp.int32, (nrows, 128), 0) == row_idx
    row = jnp.sum(jnp.where(rowsel, h2d, 0), axis=0, keepdims=True)
    cum_in = _cumsum_lanes(row)
    lt2 = (base + cum_in) < r
    col_idx = jnp.sum(lt2.astype(jnp.int32))
    prev = base + jnp.max(jnp.where(lt2, cum_in, 0))
    return row_idx * 128 + col_idx, prev


def _find1_body(h_ref, r_ref, out_ref):
    r = r_ref[0, 0]
    h = jnp.sum(h_ref[...], axis=0)
    b1, prev = _select_bin(h, r, BINS1 // 128)
    r2 = r - prev
    rowi = lax.broadcasted_iota(jnp.int32, (8, 128), 0)
    out_ref[...] = jnp.where(rowi == 0, b1, r2)


def _find2_body(h_ref, br_ref, out_ref):
    b1 = br_ref[0, 0]
    r2 = br_ref[1, 0]
    h = jnp.sum(h_ref[...], axis=0)
    b2, prev = _select_bin(h, r2, BINS2 // 128)
    b12 = b1 * BINS2 + b2
    r3 = r2 - prev
    rowi = lax.broadcasted_iota(jnp.int32, (8, 128), 0)
    out_ref[...] = jnp.where(rowi == 0, b12, r3)


def _find3_body(h_ref, br_ref, out_ref):
    b12 = br_ref[0, 0]
    r3 = br_ref[1, 0]
    h = jnp.sum(h_ref[...], axis=0)
    cum = _cumsum_lanes(h)
    b3 = jnp.sum((cum < r3).astype(jnp.int32))
    tbits = b12 * BINS3 + b3
    out_ref[...] = jnp.full((1, 1), lax.bitcast_convert_type(tbits, jnp.float32))


def _mask_body(x_ref, t_ref, out_ref):
    t = t_ref[0, 0]
    xv = x_ref[...]
    out_ref[...] = jnp.where(jnp.abs(xv) > t, xv, 0.0)


_find1 = pl.pallas_call(
    _find1_body, out_shape=jax.ShapeDtypeStruct((8, 128), jnp.int32))
_find2 = pl.pallas_call(
    _find2_body, out_shape=jax.ShapeDtypeStruct((8, 128), jnp.int32))
_find3 = pl.pallas_call(
    _find3_body, out_shape=jax.ShapeDtypeStruct((1, 1), jnp.float32))

_ROWS_PER_BLOCK = 256

_apply_mask = pl.pallas_call(
    _mask_body,
    grid=(4096 // _ROWS_PER_BLOCK,),
    in_specs=[
        pl.BlockSpec((_ROWS_PER_BLOCK, 4096), lambda i: (i, 0)),
        pl.BlockSpec((1, 1), lambda i: (0, 0)),
    ],
    out_specs=pl.BlockSpec((_ROWS_PER_BLOCK, 4096), lambda i: (i, 0)),
    out_shape=jax.ShapeDtypeStruct((4096, 4096), jnp.float32),
)


def kernel(x, sparsity, mask):
    del mask
    xi = x.reshape(-1)
    s = sparsity.reshape(())
    idx = jnp.clip((s * N - 1.0).astype(jnp.int32), 0, N - 1)
    r = (idx + 1).reshape(1, 1)
    h1 = _hist1(xi)
    br1 = _find1(h1.reshape(NW, BINS1 // 128, 128), r)
    h2 = _hist2(xi, br1.reshape(-1))
    br2 = _find2(h2.reshape(NW, BINS2 // 128, 128), br1)
    h3 = _hist3(xi, br2.reshape(-1))
    thr = _find3(h3.reshape(NW, 1, BINS3), br2)
    return _apply_mask(x, thr)


# speedup vs baseline: 76.9741x; 2.9333x over previous
import functools

import jax
import jax.numpy as jnp
from jax import lax
from jax.experimental import pallas as pl
from jax.experimental.pallas import tpu as pltpu
from jax.experimental.pallas import tpu_sc as plsc

N = 4096 * 4096
NW = 32
CHUNK = N // NW
PIECE = 16384
NPIECES = CHUNK // PIECE
UNROLL = 8
BINS1 = 1 << 12
BINS2 = 1 << 12
BINS3 = 1 << 7

_mesh = plsc.VectorSubcoreMesh(core_axis_name="c", subcore_axis_name="s")
_sc_params = pltpu.CompilerParams(needs_layout_passes=False)


def _worker_id():
    return lax.axis_index("s") * 2 + lax.axis_index("c")


def _make_hist_kernel(nbins, bin_fn, masked):
    stride = nbins + 1
    scratch = [
        pltpu.VMEM((PIECE,), jnp.float32),
        pltpu.VMEM((16 * stride,), jnp.int32),
        pltpu.VMEM((nbins,), jnp.int32),
    ]
    if masked:
        scratch.append(pltpu.VMEM((16,), jnp.int32))

    def body(x_hbm, *rest):
        if masked:
            bsel_hbm, out_hbm, data_v, hist_v, merged_v, b_v = rest
        else:
            out_hbm, data_v, hist_v, merged_v = rest
        wid = _worker_id()

        zeros = jnp.zeros((16,), jnp.int32)

        @plsc.parallel_loop(0, 16 * stride // 16, unroll=8)
        def _(i):
            hist_v[pl.ds(i * 16, 16)] = zeros

        if masked:
            pltpu.sync_copy(bsel_hbm.at[pl.ds(0, 16)], b_v)
            bvec = b_v[...]
        ones = jnp.full((16,), 1, jnp.int32)
        lane = lax.broadcasted_iota(jnp.int32, (16,), 0)
        lane_off = lane * stride

        def piece_body(p, carry):
            base = pl.multiple_of(wid * CHUNK + p * PIECE, PIECE)
            pltpu.sync_copy(x_hbm.at[pl.ds(base, PIECE)], data_v)

            @plsc.parallel_loop(0, PIECE // 16, unroll=UNROLL)
            def _(i):
                v = data_v[pl.ds(i * 16, 16)]
                bits = plsc.bitcast(v, jnp.int32) & jnp.int32(0x7FFFFFFF)
                b, key = bin_fn(bits)
                addr = b + lane_off
                if masked:
                    plsc.addupdate_scatter(hist_v, [addr], ones,
                                           mask=key == bvec)
                else:
                    plsc.addupdate_scatter(hist_v, [addr], ones)

            return carry

        lax.fori_loop(0, NPIECES, piece_body, 0)

        @plsc.parallel_loop(0, nbins // 16, unroll=2)
        def _(i):
            acc = hist_v[pl.ds(i * 16, 16)]
            for l in range(1, 16):
                acc = acc + hist_v[pl.ds(l * stride + i * 16, 16)]
            merged_v[pl.ds(i * 16, 16)] = acc

        pltpu.sync_copy(merged_v, out_hbm.at[wid])

    return pl.kernel(
        body,
        out_type=jax.ShapeDtypeStruct((NW, nbins), jnp.int32),
        mesh=_mesh,
        scratch_types=scratch,
        compiler_params=_sc_params,
    )


_hist1 = _make_hist_kernel(
    BINS1, lambda bits: (bits >> 19, None), masked=False)
_hist2 = _make_hist_kernel(
    BINS2, lambda bits: ((bits >> 7) & jnp.int32(0xFFF), bits >> 19),
    masked=True)
_hist3 = _make_hist_kernel(
    BINS3, lambda bits: (bits & jnp.int32(0x7F), bits >> 7), masked=True)


def _cumsum_rows(c):
    nrows = c.shape[0]
    k = 1
    while k < nrows:
        shifted = jnp.concatenate(
            [jnp.zeros((k, 1), jnp.int32), c[:-k]], axis=0)
        c = c + shifted
        k *= 2
    return c


def _cumsum_lanes(c):
    k = 1
    while k < 128:
        shifted = jnp.concatenate(
            [jnp.zeros((1, k), jnp.int32), c[:, :-k]], axis=1)
        c = c + shifted
        k *= 2
    return c


def _select_bin(h2d, r, nrows):
    rows = jnp.sum(h2d, axis=1, keepdims=True)
    cum_rows = _cumsum_rows(rows)
    lt = cum_rows < r
    row_idx = jnp.sum(lt.astype(jnp.int32))
    base = jnp.max(jnp.where(lt, cum_rows, 0))
    rowsel = lax.broadcasted_iota(jnp.int32, (nrows, 128), 0) == row_idx
    row = jnp.sum(jnp.where(rowsel, h2d, 0), axis=0, keepdims=True)
    cum_in = _cumsum_lanes(row)
    lt2 = (base + cum_in) < r
    col_idx = jnp.sum(lt2.astype(jnp.int32))
    prev = base + jnp.max(jnp.where(lt2, cum_in, 0))
    return row_idx * 128 + col_idx, prev


def _find1_body(h_ref, r_ref, out_ref):
    r = r_ref[0, 0]
    h = jnp.sum(h_ref[...], axis=0)
    b1, prev = _select_bin(h, r, BINS1 // 128)
    r2 = r - prev
    rowi = lax.broadcasted_iota(jnp.int32, (8, 128), 0)
    out_ref[...] = jnp.where(rowi == 0, b1, r2)


def _find2_body(h_ref, br_ref, out_ref):
    b1 = br_ref[0, 0]
    r2 = br_ref[1, 0]
    h = jnp.sum(h_ref[...], axis=0)
    b2, prev = _select_bin(h, r2, BINS2 // 128)
    b12 = b1 * BINS2 + b2
    r3 = r2 - prev
    rowi = lax.broadcasted_iota(jnp.int32, (8, 128), 0)
    out_ref[...] = jnp.where(rowi == 0, b12, r3)


def _find3_body(h_ref, br_ref, out_ref):
    b12 = br_ref[0, 0]
    r3 = br_ref[1, 0]
    h = jnp.sum(h_ref[...], axis=0)
    cum = _cumsum_lanes(h)
    b3 = jnp.sum((cum < r3).astype(jnp.int32))
    tbits = b12 * BINS3 + b3
    out_ref[...] = jnp.full((1, 1), lax.bitcast_convert_type(tbits, jnp.float32))


def _mask_body(x_ref, t_ref, out_ref):
    t = t_ref[0, 0]
    xv = x_ref[...]
    out_ref[...] = jnp.where(jnp.abs(xv) > t, xv, 0.0)


_find1 = pl.pallas_call(
    _find1_body, out_shape=jax.ShapeDtypeStruct((8, 128), jnp.int32))
_find2 = pl.pallas_call(
    _find2_body, out_shape=jax.ShapeDtypeStruct((8, 128), jnp.int32))
_find3 = pl.pallas_call(
    _find3_body, out_shape=jax.ShapeDtypeStruct((1, 1), jnp.float32))

_ROWS_PER_BLOCK = 256

_apply_mask = pl.pallas_call(
    _mask_body,
    grid=(4096 // _ROWS_PER_BLOCK,),
    in_specs=[
        pl.BlockSpec((_ROWS_PER_BLOCK, 4096), lambda i: (i, 0)),
        pl.BlockSpec((1, 1), lambda i: (0, 0)),
    ],
    out_specs=pl.BlockSpec((_ROWS_PER_BLOCK, 4096), lambda i: (i, 0)),
    out_shape=jax.ShapeDtypeStruct((4096, 4096), jnp.float32),
)


def kernel(x, sparsity, mask):
    del mask
    xi = x.reshape(-1)
    s = sparsity.reshape(())
    idx = jnp.clip((s * N - 1.0).astype(jnp.int32), 0, N - 1)
    r = (idx + 1).reshape(1, 1)
    h1 = _hist1(xi)
    br1 = _find1(h1.reshape(NW, BINS1 // 128, 128), r)
    h2 = _hist2(xi, br1.reshape(-1))
    br2 = _find2(h2.reshape(NW, BINS2 // 128, 128), br1)
    h3 = _hist3(xi, br2.reshape(-1))
    thr = _find3(h3.reshape(NW, 1, BINS3), br2)
    return _apply_mask(x, thr)


# speedup vs baseline: 95.2705x; 1.2377x over previous
import functools

import jax
import jax.numpy as jnp
from jax import lax
from jax.experimental import pallas as pl
from jax.experimental.pallas import tpu as pltpu
from jax.experimental.pallas import tpu_sc as plsc

N = 4096 * 4096
NROWS = 4096
NCOLS = 4096
NW = 32
ROWS_PER_W = NROWS // NW
PIECE_ROWS = 8
PIECE = PIECE_ROWS * NCOLS
NPIECES = ROWS_PER_W // PIECE_ROWS
UNROLL = 8
BINS1 = 1 << 12
BINS2 = 1 << 12
BINS3 = 1 << 7

_mesh = plsc.VectorSubcoreMesh(core_axis_name="c", subcore_axis_name="s")
_sc_params = pltpu.CompilerParams(needs_layout_passes=False)


def _worker_id():
    return lax.axis_index("s") * 2 + lax.axis_index("c")


def _make_hist_kernel(nbins, bin_fn, masked):
    stride = nbins + 1
    scratch = [
        pltpu.VMEM((PIECE_ROWS, NCOLS), jnp.float32),
        pltpu.VMEM((16 * stride,), jnp.int32),
        pltpu.VMEM((nbins,), jnp.int32),
    ]
    if masked:
        scratch.append(pltpu.VMEM((16,), jnp.int32))

    def body(x_hbm, *rest):
        if masked:
            bsel_hbm, out_hbm, data_v, hist_v, merged_v, b_v = rest
        else:
            out_hbm, data_v, hist_v, merged_v = rest
        wid = _worker_id()

        zeros = jnp.zeros((16,), jnp.int32)

        @plsc.parallel_loop(0, 16 * stride // 16, unroll=8)
        def _(i):
            hist_v[pl.ds(i * 16, 16)] = zeros

        if masked:
            pltpu.sync_copy(bsel_hbm.at[pl.ds(0, 16)], b_v)
            bvec = b_v[...]
        ones = jnp.full((16,), 1, jnp.int32)
        lane = lax.broadcasted_iota(jnp.int32, (16,), 0)
        lane_off = lane * stride

        def piece_body(p, carry):
            row0 = pl.multiple_of(wid * ROWS_PER_W + p * PIECE_ROWS,
                                  PIECE_ROWS)
            pltpu.sync_copy(x_hbm.at[pl.ds(row0, PIECE_ROWS)], data_v)

            for rr in range(PIECE_ROWS):
                @plsc.parallel_loop(0, NCOLS // 16, unroll=UNROLL)
                def _(i, rr=rr):
                    v = data_v[rr, pl.ds(i * 16, 16)]
                    bits = plsc.bitcast(v, jnp.int32) & jnp.int32(0x7FFFFFFF)
                    b, key = bin_fn(bits)
                    addr = b + lane_off
                    if masked:
                        plsc.addupdate_scatter(hist_v, [addr], ones,
                                               mask=key == bvec)
                    else:
                        plsc.addupdate_scatter(hist_v, [addr], ones)

            return carry

        lax.fori_loop(0, NPIECES, piece_body, 0)

        @plsc.parallel_loop(0, nbins // 16, unroll=2)
        def _(i):
            acc = hist_v[pl.ds(i * 16, 16)]
            for l in range(1, 16):
                acc = acc + hist_v[pl.ds(l * stride + i * 16, 16)]
            merged_v[pl.ds(i * 16, 16)] = acc

        pltpu.sync_copy(merged_v, out_hbm.at[wid])

    return pl.kernel(
        body,
        out_type=jax.ShapeDtypeStruct((NW, nbins), jnp.int32),
        mesh=_mesh,
        scratch_types=scratch,
        compiler_params=_sc_params,
    )


_hist1 = _make_hist_kernel(
    BINS1, lambda bits: (bits >> 19, None), masked=False)
_hist2 = _make_hist_kernel(
    BINS2, lambda bits: ((bits >> 7) & jnp.int32(0xFFF), bits >> 19),
    masked=True)
_hist3 = _make_hist_kernel(
    BINS3, lambda bits: (bits & jnp.int32(0x7F), bits >> 7), masked=True)


def _cumsum_rows(c):
    nrows = c.shape[0]
    k = 1
    while k < nrows:
        shifted = jnp.concatenate(
            [jnp.zeros((k, 1), jnp.int32), c[:-k]], axis=0)
        c = c + shifted
        k *= 2
    return c


def _cumsum_lanes(c):
    k = 1
    while k < 128:
        shifted = jnp.concatenate(
            [jnp.zeros((1, k), jnp.int32), c[:, :-k]], axis=1)
        c = c + shifted
        k *= 2
    return c


def _select_bin(h2d, r, nrows):
    rows = jnp.sum(h2d, axis=1, keepdims=True)
    cum_rows = _cumsum_rows(rows)
    lt = cum_rows < r
    row_idx = jnp.sum(lt.astype(jnp.int32))
    base = jnp.max(jnp.where(lt, cum_rows, 0))
    rowsel = lax.broadcasted_iota(jnp.int32, (nrows, 128), 0) == row_idx
    row = jnp.sum(jnp.where(rowsel, h2d, 0), axis=0, keepdims=True)
    cum_in = _cumsum_lanes(row)
    lt2 = (base + cum_in) < r
    col_idx = jnp.sum(lt2.astype(jnp.int32))
    prev = base + jnp.max(jnp.where(lt2, cum_in, 0))
    return row_idx * 128 + col_idx, prev


def _find1_body(h_ref, r_ref, out_ref):
    r = r_ref[0, 0]
    h = jnp.sum(h_ref[...], axis=0)
    b1, prev = _select_bin(h, r, BINS1 // 128)
    r2 = r - prev
    rowi = lax.broadcasted_iota(jnp.int32, (8, 128), 0)
    out_ref[...] = jnp.where(rowi == 0, b1, r2)


def _find2_body(h_ref, br_ref, out_ref):
    b1 = br_ref[0, 0]
    r2 = br_ref[1, 0]
    h = jnp.sum(h_ref[...], axis=0)
    b2, prev = _select_bin(h, r2, BINS2 // 128)
    b12 = b1 * BINS2 + b2
    r3 = r2 - prev
    rowi = lax.broadcasted_iota(jnp.int32, (8, 128), 0)
    out_ref[...] = jnp.where(rowi == 0, b12, r3)


def _find3_body(h_ref, br_ref, out_ref):
    b12 = br_ref[0, 0]
    r3 = br_ref[1, 0]
    h = jnp.sum(h_ref[...], axis=0)
    cum = _cumsum_lanes(h)
    b3 = jnp.sum((cum < r3).astype(jnp.int32))
    tbits = b12 * BINS3 + b3
    out_ref[...] = jnp.full((1, 1), lax.bitcast_convert_type(tbits, jnp.float32))


def _mask_body(x_ref, t_ref, out_ref):
    t = t_ref[0, 0]
    xv = x_ref[...]
    out_ref[...] = jnp.where(jnp.abs(xv) > t, xv, 0.0)


_find1 = pl.pallas_call(
    _find1_body, out_shape=jax.ShapeDtypeStruct((8, 128), jnp.int32))
_find2 = pl.pallas_call(
    _find2_body, out_shape=jax.ShapeDtypeStruct((8, 128), jnp.int32))
_find3 = pl.pallas_call(
    _find3_body, out_shape=jax.ShapeDtypeStruct((1, 1), jnp.float32))

_ROWS_PER_BLOCK = 256

_apply_mask = pl.pallas_call(
    _mask_body,
    grid=(4096 // _ROWS_PER_BLOCK,),
    in_specs=[
        pl.BlockSpec((_ROWS_PER_BLOCK, 4096), lambda i: (i, 0)),
        pl.BlockSpec((1, 1), lambda i: (0, 0)),
    ],
    out_specs=pl.BlockSpec((_ROWS_PER_BLOCK, 4096), lambda i: (i, 0)),
    out_shape=jax.ShapeDtypeStruct((4096, 4096), jnp.float32),
)


def kernel(x, sparsity, mask):
    del mask
    xi = x
    s = sparsity.reshape(())
    idx = jnp.clip((s * N - 1.0).astype(jnp.int32), 0, N - 1)
    r = (idx + 1).reshape(1, 1)
    h1 = _hist1(xi)
    br1 = _find1(h1.reshape(NW, BINS1 // 128, 128), r)
    h2 = _hist2(xi, br1.reshape(-1))
    br2 = _find2(h2.reshape(NW, BINS2 // 128, 128), br1)
    h3 = _hist3(xi, br2.reshape(-1))
    thr = _find3(h3.reshape(NW, 1, BINS3), br2)
    return _apply_mask(x, thr)


# speedup vs baseline: 139.9048x; 1.4685x over previous
import functools

import jax
import jax.numpy as jnp
from jax import lax
from jax.experimental import pallas as pl
from jax.experimental.pallas import tpu as pltpu
from jax.experimental.pallas import tpu_sc as plsc

N = 4096 * 4096
NROWS = 4096
NCOLS = 4096
NW = 32
ROWS_PER_W = NROWS // NW
PIECE_ROWS = 8
PIECE = PIECE_ROWS * NCOLS
NPIECES = ROWS_PER_W // PIECE_ROWS
UNROLL = 8
BINS1 = 1 << 12
BINS2 = 1 << 12
BINS3 = 1 << 7

_mesh = plsc.VectorSubcoreMesh(core_axis_name="c", subcore_axis_name="s")
_sc_params = pltpu.CompilerParams(needs_layout_passes=False)


def _worker_id():
    return lax.axis_index("s") * 2 + lax.axis_index("c")


def _make_hist_kernel(nbins, bin_fn, masked):
    stride = nbins + 1
    nbanks = 15
    hist_words = (nbanks * stride + 15) // 16 * 16
    scratch = [
        pltpu.VMEM((PIECE_ROWS, NCOLS), jnp.float32),
        pltpu.VMEM((PIECE_ROWS, NCOLS), jnp.float32),
        pltpu.VMEM((hist_words,), jnp.int32),
        pltpu.SemaphoreType.DMA,
        pltpu.SemaphoreType.DMA,
    ]
    if masked:
        scratch.append(pltpu.VMEM((16,), jnp.int32))

    def body(x_hbm, *rest):
        if masked:
            bsel_hbm, out_hbm, data0, data1, hist_v, sem0, sem1, b_v = rest
        else:
            out_hbm, data0, data1, hist_v, sem0, sem1 = rest
        wid = _worker_id()
        rowbase = wid * ROWS_PER_W

        def start(p, buf, sem):
            row0 = pl.multiple_of(rowbase + p * PIECE_ROWS, PIECE_ROWS)
            return pltpu.async_copy(x_hbm.at[pl.ds(row0, PIECE_ROWS)],
                                    buf, sem)

        def wait(p, buf, sem):
            row0 = pl.multiple_of(rowbase + p * PIECE_ROWS, PIECE_ROWS)
            pltpu.make_async_copy(x_hbm.at[pl.ds(row0, PIECE_ROWS)],
                                  buf, sem).wait()

        start(0, data0, sem0)

        zeros = jnp.zeros((16,), jnp.int32)

        @plsc.parallel_loop(0, hist_words // 16, unroll=8)
        def _(i):
            hist_v[pl.ds(i * 16, 16)] = zeros

        if masked:
            pltpu.sync_copy(bsel_hbm.at[pl.ds(0, 16)], b_v)
            bvec = b_v[...]
        ones = jnp.full((16,), 1, jnp.int32)
        lane = lax.broadcasted_iota(jnp.int32, (16,), 0)
        lane_off = jnp.minimum(lane, nbanks - 1) * stride

        def process(buf):
            @plsc.parallel_loop(0, PIECE // 16, unroll=UNROLL)
            def _(i):
                rr = i >> 8
                cc = (i & 255) * 16
                v = buf[rr, pl.ds(cc, 16)]
                bits = plsc.bitcast(v, jnp.int32) & jnp.int32(0x7FFFFFFF)
                b, key = bin_fn(bits)
                addr = b + lane_off
                if masked:
                    plsc.addupdate_scatter(hist_v, [addr], ones,
                                           mask=key == bvec)
                else:
                    plsc.addupdate_scatter(hist_v, [addr], ones)

        def pair_body(q, carry):
            pa = 2 * q
            start(pa + 1, data1, sem1)
            wait(pa, data0, sem0)
            process(data0)

            @pl.when(q < NPIECES // 2 - 1)
            def _():
                start(pa + 2, data0, sem0)

            wait(pa + 1, data1, sem1)
            process(data1)
            return carry

        lax.fori_loop(0, NPIECES // 2, pair_body, 0)

        @plsc.parallel_loop(0, nbins // 16, unroll=2)
        def _(i):
            acc = hist_v[pl.ds(i * 16, 16)]
            for l in range(1, nbanks):
                acc = acc + hist_v[pl.ds(l * stride + i * 16, 16)]
            hist_v[pl.ds(i * 16, 16)] = acc

        pltpu.sync_copy(hist_v.at[pl.ds(0, nbins)], out_hbm.at[wid])

    return pl.kernel(
        body,
        out_type=jax.ShapeDtypeStruct((NW, nbins), jnp.int32),
        mesh=_mesh,
        scratch_types=scratch,
        compiler_params=_sc_params,
    )


_hist1 = _make_hist_kernel(
    BINS1, lambda bits: (bits >> 19, None), masked=False)
_hist2 = _make_hist_kernel(
    BINS2, lambda bits: ((bits >> 7) & jnp.int32(0xFFF), bits >> 19),
    masked=True)
_hist3 = _make_hist_kernel(
    BINS3, lambda bits: (bits & jnp.int32(0x7F), bits >> 7), masked=True)


def _cumsum_rows(c):
    nrows = c.shape[0]
    k = 1
    while k < nrows:
        shifted = jnp.concatenate(
            [jnp.zeros((k, 1), jnp.int32), c[:-k]], axis=0)
        c = c + shifted
        k *= 2
    return c


def _cumsum_lanes(c):
    k = 1
    while k < 128:
        shifted = jnp.concatenate(
            [jnp.zeros((1, k), jnp.int32), c[:, :-k]], axis=1)
        c = c + shifted
        k *= 2
    return c


def _select_bin(h2d, r, nrows):
    rows = jnp.sum(h2d, axis=1, keepdims=True)
    cum_rows = _cumsum_rows(rows)
    lt = cum_rows < r
    row_idx = jnp.sum(lt.astype(jnp.int32))
    base = jnp.max(jnp.where(lt, cum_rows, 0))
    rowsel = lax.broadcasted_iota(jnp.int32, (nrows, 128), 0) == row_idx
    row = jnp.sum(jnp.where(rowsel, h2d, 0), axis=0, keepdims=True)
    cum_in = _cumsum_lanes(row)
    lt2 = (base + cum_in) < r
    col_idx = jnp.sum(lt2.astype(jnp.int32))
    prev = base + jnp.max(jnp.where(lt2, cum_in, 0))
    return row_idx * 128 + col_idx, prev


def _find1_body(h_ref, r_ref, out_ref):
    r = r_ref[0, 0]
    h = jnp.sum(h_ref[...], axis=0)
    b1, prev = _select_bin(h, r, BINS1 // 128)
    r2 = r - prev
    rowi = lax.broadcasted_iota(jnp.int32, (8, 128), 0)
    out_ref[...] = jnp.where(rowi == 0, b1, r2)


def _find2_body(h_ref, br_ref, out_ref):
    b1 = br_ref[0, 0]
    r2 = br_ref[1, 0]
    h = jnp.sum(h_ref[...], axis=0)
    b2, prev = _select_bin(h, r2, BINS2 // 128)
    b12 = b1 * BINS2 + b2
    r3 = r2 - prev
    rowi = lax.broadcasted_iota(jnp.int32, (8, 128), 0)
    out_ref[...] = jnp.where(rowi == 0, b12, r3)


def _find3_body(h_ref, br_ref, out_ref):
    b12 = br_ref[0, 0]
    r3 = br_ref[1, 0]
    h = jnp.sum(h_ref[...], axis=0)
    cum = _cumsum_lanes(h)
    b3 = jnp.sum((cum < r3).astype(jnp.int32))
    tbits = b12 * BINS3 + b3
    out_ref[...] = jnp.full((1, 1), lax.bitcast_convert_type(tbits, jnp.float32))


def _mask_body(x_ref, t_ref, out_ref):
    t = t_ref[0, 0]
    xv = x_ref[...]
    out_ref[...] = jnp.where(jnp.abs(xv) > t, xv, 0.0)


_find1 = pl.pallas_call(
    _find1_body, out_shape=jax.ShapeDtypeStruct((8, 128), jnp.int32))
_find2 = pl.pallas_call(
    _find2_body, out_shape=jax.ShapeDtypeStruct((8, 128), jnp.int32))
_find3 = pl.pallas_call(
    _find3_body, out_shape=jax.ShapeDtypeStruct((1, 1), jnp.float32))

_ROWS_PER_BLOCK = 256

_apply_mask = pl.pallas_call(
    _mask_body,
    grid=(4096 // _ROWS_PER_BLOCK,),
    in_specs=[
        pl.BlockSpec((_ROWS_PER_BLOCK, 4096), lambda i: (i, 0)),
        pl.BlockSpec((1, 1), lambda i: (0, 0)),
    ],
    out_specs=pl.BlockSpec((_ROWS_PER_BLOCK, 4096), lambda i: (i, 0)),
    out_shape=jax.ShapeDtypeStruct((4096, 4096), jnp.float32),
)


def kernel(x, sparsity, mask):
    del mask
    xi = x
    s = sparsity.reshape(())
    idx = jnp.clip((s * N - 1.0).astype(jnp.int32), 0, N - 1)
    r = (idx + 1).reshape(1, 1)
    h1 = _hist1(xi)
    br1 = _find1(h1.reshape(NW, BINS1 // 128, 128), r)
    h2 = _hist2(xi, br1.reshape(-1))
    br2 = _find2(h2.reshape(NW, BINS2 // 128, 128), br1)
    h3 = _hist3(xi, br2.reshape(-1))
    thr = _find3(h3.reshape(NW, 1, BINS3), br2)
    return _apply_mask(x, thr)
